# baseline (speedup 1.0000x reference)
import functools

import jax
import jax.numpy as jnp
from jax import lax
from jax.experimental import pallas as pl
from jax.experimental.pallas import tpu as pltpu
from jax.experimental.pallas import tpu_sc as plsc

_SC_CORES = 2
_SC_SUBCORES = 16


def _make_pool_kernel(V, D, B, L):
    nc, ns = _SC_CORES, _SC_SUBCORES
    nw = nc * ns
    assert B % nw == 0
    b_per_w = B // nw
    nvec = D // 16
    mesh = plsc.VectorSubcoreMesh(
        core_axis_name="c", subcore_axis_name="s", num_cores=nc, num_subcores=ns
    )
    npairs = b_per_w // 2

    @functools.partial(
        pl.kernel,
        out_type=jax.ShapeDtypeStruct((B, D), jnp.float32),
        mesh=mesh,
        scratch_types=[
            pltpu.VMEM((b_per_w, L), jnp.int32),
            pltpu.VMEM((L, D), jnp.float32),
            pltpu.VMEM((L, D), jnp.float32),
            pltpu.VMEM((b_per_w, D), jnp.float32),
            pltpu.SemaphoreType.DMA,
            pltpu.SemaphoreType.DMA,
        ],
        compiler_params=pltpu.CompilerParams(use_tc_tiling_on_sc=False),
    )
    def pool(x_hbm, table_hbm, out_hbm, idx_v, rows0_v, rows1_v, h_v, sem0, sem1):
        wid = lax.axis_index("s") * nc + lax.axis_index("c")
        base = wid * b_per_w
        pltpu.sync_copy(x_hbm.at[pl.ds(base, b_per_w)], idx_v)

        def start(e, buf, sem):
            pltpu.async_copy(table_hbm.at[idx_v.at[e]], buf, sem)

        def wait(buf, sem):
            pltpu.make_async_copy(table_hbm.at[idx_v.at[0]], buf, sem).wait()

        def reduce_into(buf, e):
            def red(r, accs):
                new = []
                for j in range(nvec):
                    a = accs[j]
                    for k in range(4):
                        a = a + buf[r * 4 + k, pl.ds(j * 16, 16)]
                    new.append(a)
                return tuple(new)

            accs = tuple(jnp.zeros((16,), jnp.float32) for _ in range(nvec))
            accs = lax.fori_loop(0, L // 4, red, accs)
            scale = jnp.float32(1.0 / L)
            for j in range(nvec):
                h_v[e, pl.ds(j * 16, 16)] = accs[j] * scale

        start(0, rows0_v, sem0)

        def pair(g, carry):
            e0 = g * 2
            start(e0 + 1, rows1_v, sem1)
            wait(rows0_v, sem0)
            reduce_into(rows0_v, e0)

            @pl.when(g < npairs - 1)
            def _():
                start(e0 + 2, rows0_v, sem0)

            wait(rows1_v, sem1)
            reduce_into(rows1_v, e0 + 1)
            return carry

        lax.fori_loop(0, npairs, pair, 0)
        pltpu.sync_copy(h_v, out_hbm.at[pl.ds(base, b_per_w)])

    return pool


def _make_weff_kernel(D, V, R, TV):
    nblk = pl.cdiv(V, TV)

    def body(w_ref, b_ref, a_ref, bl_ref, out_ref):
        w_eff = w_ref[...] + jnp.dot(
            a_ref[...], bl_ref[...], preferred_element_type=jnp.float32
        )
        out_ref[...] = jnp.concatenate([w_eff, b_ref[...]], axis=0)

    return pl.pallas_call(
        body,
        grid=(nblk,),
        in_specs=[
            pl.BlockSpec((D, TV), lambda i: (0, i)),
            pl.BlockSpec((1, TV), lambda i: (0, i)),
            pl.BlockSpec((D, R), lambda i: (0, 0)),
            pl.BlockSpec((R, TV), lambda i: (0, i)),
        ],
        out_specs=pl.BlockSpec((D + 1, TV), lambda i: (0, i)),
        out_shape=jax.ShapeDtypeStruct((D + 1, V), jnp.float32),
    )


def _make_proj_kernel(B, D, V, TV):
    nblk = pl.cdiv(V, TV)

    def body(h_ref, w_ref, out_ref):
        out_ref[...] = jax.lax.dot_general(
            w_ref[...],
            h_ref[...],
            (((0,), (0,)), ((), ())),
            preferred_element_type=jnp.float32,
        )

    return pl.pallas_call(
        body,
        grid=(nblk,),
        in_specs=[
            pl.BlockSpec((D + 1, B), lambda i: (0, 0)),
            pl.BlockSpec((D + 1, TV), lambda i: (0, i)),
        ],
        out_specs=pl.BlockSpec((TV, B), lambda i: (i, 0)),
        out_shape=jax.ShapeDtypeStruct((V, B), jnp.float32),
    )


@jax.jit
def kernel(x, embed_table, W, b, A, B_lora):
    B, L = x.shape
    V, D = embed_table.shape
    R = A.shape[1]

    pool = _make_pool_kernel(V, D, B, L)
    h = pool(x, embed_table)
    w_aug = _make_weff_kernel(D, V, R, TV=4096)(W, b.reshape(1, V), A, B_lora)
    h_aug = jnp.concatenate([h.T, jnp.ones((1, B), jnp.float32)], axis=0)

    proj = _make_proj_kernel(B, D, V, TV=4096)
    out_t = proj(h_aug, w_aug)
    return jnp.swapaxes(out_t, 0, 1)


# speedup vs baseline: 1.0387x; 1.0387x over previous
import functools

import jax
import jax.numpy as jnp
from jax import lax
from jax.experimental import pallas as pl
from jax.experimental.pallas import tpu as pltpu
from jax.experimental.pallas import tpu_sc as plsc

_SC_CORES = 2
_SC_SUBCORES = 16


def _make_pool_kernel(V, D, B, L):
    nc, ns = _SC_CORES, _SC_SUBCORES
    nw = nc * ns
    assert B % nw == 0
    b_per_w = B // nw
    nvec = D // 16
    mesh = plsc.VectorSubcoreMesh(
        core_axis_name="c", subcore_axis_name="s", num_cores=nc, num_subcores=ns
    )
    npairs = b_per_w // 2

    @functools.partial(
        pl.kernel,
        out_type=jax.ShapeDtypeStruct((B, D), jnp.float32),
        mesh=mesh,
        scratch_types=[
            pltpu.VMEM((b_per_w, L), jnp.int32),
            pltpu.VMEM((L, D), jnp.float32),
            pltpu.VMEM((L, D), jnp.float32),
            pltpu.VMEM((b_per_w, D), jnp.float32),
            pltpu.SemaphoreType.DMA,
            pltpu.SemaphoreType.DMA,
        ],
        compiler_params=pltpu.CompilerParams(use_tc_tiling_on_sc=False),
    )
    def pool(x_hbm, table_hbm, out_hbm, idx_v, rows0_v, rows1_v, h_v, sem0, sem1):
        wid = lax.axis_index("s") * nc + lax.axis_index("c")
        base = wid * b_per_w
        pltpu.sync_copy(x_hbm.at[pl.ds(base, b_per_w)], idx_v)

        def start(e, buf, sem):
            pltpu.async_copy(table_hbm.at[idx_v.at[e]], buf, sem)

        def wait(buf, sem):
            pltpu.make_async_copy(table_hbm.at[idx_v.at[0]], buf, sem).wait()

        def reduce_into(buf, e):
            def red(r, accs):
                new = []
                for j in range(nvec):
                    a = accs[j]
                    for k in range(4):
                        a = a + buf[r * 4 + k, pl.ds(j * 16, 16)]
                    new.append(a)
                return tuple(new)

            accs = tuple(jnp.zeros((16,), jnp.float32) for _ in range(nvec))
            accs = lax.fori_loop(0, L // 4, red, accs)
            scale = jnp.float32(1.0 / L)
            for j in range(nvec):
                h_v[e, pl.ds(j * 16, 16)] = accs[j] * scale

        start(0, rows0_v, sem0)

        def pair(g, carry):
            e0 = g * 2
            start(e0 + 1, rows1_v, sem1)
            wait(rows0_v, sem0)
            reduce_into(rows0_v, e0)

            @pl.when(g < npairs - 1)
            def _():
                start(e0 + 2, rows0_v, sem0)

            wait(rows1_v, sem1)
            reduce_into(rows1_v, e0 + 1)
            return carry

        lax.fori_loop(0, npairs, pair, 0)
        pltpu.sync_copy(h_v, out_hbm.at[pl.ds(base, b_per_w)])

    return pool


def _make_proj_kernel(B, D, V, R, TV):
    nblk = pl.cdiv(V, TV)

    def body(h_ref, w_ref, b_ref, a_ref, bl_ref, out_ref):
        w_eff = w_ref[...] + jnp.dot(
            a_ref[...], bl_ref[...], preferred_element_type=jnp.float32
        )
        w_aug = jnp.concatenate([w_eff, b_ref[...]], axis=0)
        out_ref[...] = jax.lax.dot_general(
            w_aug,
            h_ref[...],
            (((0,), (0,)), ((), ())),
            preferred_element_type=jnp.float32,
        )

    return pl.pallas_call(
        body,
        grid=(nblk,),
        in_specs=[
            pl.BlockSpec((D + 1, B), lambda i: (0, 0)),
            pl.BlockSpec((D, TV), lambda i: (0, i)),
            pl.BlockSpec((1, TV), lambda i: (0, i)),
            pl.BlockSpec((D, R), lambda i: (0, 0)),
            pl.BlockSpec((R, TV), lambda i: (0, i)),
        ],
        out_specs=pl.BlockSpec((TV, B), lambda i: (i, 0)),
        out_shape=jax.ShapeDtypeStruct((V, B), jnp.float32),
    )


@jax.jit
def kernel(x, embed_table, W, b, A, B_lora):
    B, L = x.shape
    V, D = embed_table.shape
    R = A.shape[1]

    pool = _make_pool_kernel(V, D, B, L)
    h = pool(x, embed_table)
    h_aug = jnp.concatenate([h.T, jnp.ones((1, B), jnp.float32)], axis=0)

    proj = _make_proj_kernel(B, D, V, R, TV=4096)
    out_t = proj(h_aug, W, b.reshape(1, V), A, B_lora)
    return jnp.swapaxes(out_t, 0, 1)
